# baseline (speedup 1.0000x reference)
import functools

import jax
import jax.numpy as jnp
from jax import lax
from jax.experimental import pallas as pl
from jax.experimental.pallas import tpu as pltpu
from jax.experimental.pallas import tpu_sc as plsc

_NUM_WORKERS = 32
_LANES = 16


def kernel(distance, similarities):
    B, I, J, K, D = distance.shape
    N = B * I * J
    PW = N // _NUM_WORKERS
    assert PW * _NUM_WORKERS == N and PW % _LANES == 0
    HALF = PW // 2
    IPW = (I * J) // PW
    WPB = I // (PW // J)
    IROWS = I // WPB

    dist = distance.reshape(N * K, D)
    sims = similarities.reshape(N, K)

    mesh = plsc.VectorSubcoreMesh(core_axis_name="c", subcore_axis_name="s")

    @functools.partial(
        pl.kernel,
        mesh=mesh,
        compiler_params=pltpu.CompilerParams(needs_layout_passes=False),
        out_type=[
            jax.ShapeDtypeStruct((B, I, J, D), jnp.float32),
            jax.ShapeDtypeStruct((N,), jnp.int32),
        ],
        scratch_types=[
            pltpu.VMEM((PW, K), jnp.float32),
            pltpu.VMEM((PW,), jnp.int32),
            pltpu.VMEM((PW,), jnp.int32),
            pltpu.VMEM((PW, D), jnp.float32),
            pltpu.SemaphoreType.DMA,
        ],
    )
    def body(dist_hbm, sims_hbm, out_hbm, arg_hbm, sims_v, idx_v, arg_v, rows_v, sem):
        wid = lax.axis_index("s") * 2 + lax.axis_index("c")
        base = wid * PW
        b0 = wid // WPB
        i0 = (wid % WPB) * IROWS
        pltpu.sync_copy(sims_hbm.at[pl.ds(base, PW)], sims_v)

        lane = lax.iota(jnp.int32, _LANES)
        zeros = jnp.zeros((_LANES,), jnp.int32)

        def argmax_group(g, _):
            row0 = g * _LANES
            rows = row0 + lane

            def step(k, carry):
                bv, bi = carry
                col = jnp.full((_LANES,), k, jnp.int32)
                v = plsc.load_gather(sims_v, [rows, col])
                m = v > bv
                return jnp.where(m, v, bv), jnp.where(m, k, bi)

            bv0 = plsc.load_gather(sims_v, [rows, zeros])
            _, bi = lax.fori_loop(1, K, step, (bv0, zeros))
            idx_v[pl.ds(row0, _LANES)] = (base + rows) * K + bi
            arg_v[pl.ds(row0, _LANES)] = bi
            return 0

        lax.fori_loop(0, PW // _LANES, argmax_group, 0)

        copies = [
            pltpu.async_copy(
                dist_hbm.at[idx_v.at[pl.ds(j * HALF, HALF)]],
                rows_v.at[pl.ds(j * HALF, HALF)],
                sem,
            )
            for j in range(2)
        ]
        for c in copies:
            c.wait()

        for r in range(IROWS):
            pltpu.sync_copy(rows_v.at[pl.ds(r * J, J)], out_hbm.at[b0, i0 + r])
        pltpu.sync_copy(arg_v, arg_hbm.at[pl.ds(base, PW)])

    out, arg = body(dist, sims)
    return out, arg.reshape(B, I, J)


# speedup vs baseline: 1.0602x; 1.0602x over previous
import functools

import jax
import jax.numpy as jnp
from jax import lax
from jax.experimental import pallas as pl
from jax.experimental.pallas import tpu as pltpu
from jax.experimental.pallas import tpu_sc as plsc

_NUM_WORKERS = 32
_LANES = 16


def kernel(distance, similarities):
    B, I, J, K, D = distance.shape
    N = B * I * J
    PW = N // _NUM_WORKERS
    assert PW * _NUM_WORKERS == N and PW % _LANES == 0
    HALF = PW // 2
    IPW = (I * J) // PW
    WPB = I // (PW // J)
    IROWS = I // WPB

    dist = distance.reshape(N * K, D)
    sims = similarities.reshape(N, K)

    mesh = plsc.VectorSubcoreMesh(core_axis_name="c", subcore_axis_name="s")

    @functools.partial(
        pl.kernel,
        mesh=mesh,
        compiler_params=pltpu.CompilerParams(needs_layout_passes=False),
        out_type=[
            jax.ShapeDtypeStruct((B, I, J, D), jnp.float32),
            jax.ShapeDtypeStruct((N,), jnp.int32),
        ],
        scratch_types=[
            pltpu.VMEM((PW, K), jnp.float32),
            pltpu.VMEM((PW,), jnp.int32),
            pltpu.VMEM((PW,), jnp.int32),
            pltpu.VMEM((PW, D), jnp.float32),
            pltpu.SemaphoreType.DMA,
            pltpu.SemaphoreType.DMA,
        ],
    )
    def body(
        dist_hbm, sims_hbm, out_hbm, arg_hbm, sims_v, idx_v, arg_v, rows_v, sem_g, sem_w
    ):
        wid = lax.axis_index("s") * 2 + lax.axis_index("c")
        base = wid * PW
        b0 = wid // WPB
        i0 = (wid % WPB) * IROWS
        pltpu.sync_copy(sims_hbm.at[pl.ds(base, PW)], sims_v)

        lane = lax.iota(jnp.int32, _LANES)
        zeros = jnp.zeros((_LANES,), jnp.int32)

        def argmax_group(g, _):
            row0 = g * _LANES
            rows = row0 + lane
            bv = plsc.load_gather(sims_v, [rows, zeros])
            bi = zeros
            for k in range(1, K):
                col = jnp.full((_LANES,), k, jnp.int32)
                v = plsc.load_gather(sims_v, [rows, col])
                m = v > bv
                bv = jnp.where(m, v, bv)
                bi = jnp.where(m, k, bi)
            idx_v[pl.ds(row0, _LANES)] = (base + rows) * K + bi
            arg_v[pl.ds(row0, _LANES)] = bi
            return 0

        NCHUNK = 3
        CR = PW // NCHUNK
        GPC = PW // _LANES // NCHUNK
        RPC = IROWS // NCHUNK
        gathers = []
        for c in range(NCHUNK):
            lax.fori_loop(c * GPC, (c + 1) * GPC, argmax_group, 0)
            gathers.append(
                pltpu.async_copy(
                    dist_hbm.at[idx_v.at[pl.ds(c * CR, CR)]],
                    rows_v.at[pl.ds(c * CR, CR)],
                    sem_g,
                )
            )
        writes = []
        for c in range(NCHUNK):
            gathers[c].wait()
            for r in range(RPC):
                ri = c * RPC + r
                writes.append(
                    pltpu.async_copy(
                        rows_v.at[pl.ds(ri * J, J)], out_hbm.at[b0, i0 + ri], sem_w
                    )
                )
        pltpu.sync_copy(arg_v, arg_hbm.at[pl.ds(base, PW)])
        for w in writes:
            w.wait()

    out, arg = body(dist, sims)
    return out, arg.reshape(B, I, J)


# speedup vs baseline: 1.1952x; 1.1274x over previous
import functools

import jax
import jax.numpy as jnp
from jax import lax
from jax.experimental import pallas as pl
from jax.experimental.pallas import tpu as pltpu
from jax.experimental.pallas import tpu_sc as plsc

_NUM_WORKERS = 32


def kernel(distance, similarities):
    B, I, J, K, D = distance.shape
    N = B * I * J
    PW = N // _NUM_WORKERS
    assert PW * _NUM_WORKERS == N
    NCHUNK = 3
    CR = PW // NCHUNK
    WPB = _NUM_WORKERS // B
    IROWS = I // WPB
    RPC = IROWS // NCHUNK

    dist = distance.reshape(N * K, D)
    sims = similarities.reshape(N, K)

    def argmax_body(s_ref, arg_ref, idx_ref):
        s = s_ref[...]
        mx = jnp.max(s, axis=1, keepdims=True)
        kio = lax.broadcasted_iota(jnp.int32, (N, K), 1)
        bi = jnp.min(jnp.where(s == mx, kio, K), axis=1)
        arg_ref[...] = bi.reshape(B, I, J)
        idx_ref[...] = lax.iota(jnp.int32, N) * K + bi

    arg, idx = pl.pallas_call(
        argmax_body,
        out_shape=[
            jax.ShapeDtypeStruct((B, I, J), jnp.int32),
            jax.ShapeDtypeStruct((N,), jnp.int32),
        ],
    )(sims)

    mesh = plsc.VectorSubcoreMesh(core_axis_name="c", subcore_axis_name="s")

    @functools.partial(
        pl.kernel,
        mesh=mesh,
        compiler_params=pltpu.CompilerParams(needs_layout_passes=False),
        out_type=jax.ShapeDtypeStruct((B, I, J, D), jnp.float32),
        scratch_types=[
            pltpu.VMEM((PW,), jnp.int32),
            pltpu.VMEM((PW, D), jnp.float32),
            pltpu.SemaphoreType.DMA,
            pltpu.SemaphoreType.DMA,
        ],
    )
    def gather_body(dist_hbm, idx_hbm, out_hbm, idx_v, rows_v, sem_g, sem_w):
        wid = lax.axis_index("s") * 2 + lax.axis_index("c")
        base = wid * PW
        b0 = wid // WPB
        i0 = (wid % WPB) * IROWS
        pltpu.sync_copy(idx_hbm.at[pl.ds(base, PW)], idx_v)
        gathers = [
            pltpu.async_copy(
                dist_hbm.at[idx_v.at[pl.ds(c * CR, CR)]],
                rows_v.at[pl.ds(c * CR, CR)],
                sem_g,
            )
            for c in range(NCHUNK)
        ]
        writes = []
        for c in range(NCHUNK):
            gathers[c].wait()
            for r in range(RPC):
                ri = c * RPC + r
                writes.append(
                    pltpu.async_copy(
                        rows_v.at[pl.ds(ri * J, J)], out_hbm.at[b0, i0 + ri], sem_w
                    )
                )
        for w in writes:
            w.wait()

    out = gather_body(dist, idx)
    return out, arg
